# baseline (speedup 1.0000x reference)
import functools

import jax
import jax.numpy as jnp
from jax import lax
from jax.experimental import pallas as pl
from jax.experimental.pallas import tpu as pltpu
from jax.experimental.pallas import tpu_sc as plsc

_N = 10000
_E = 320000
_F = 128
_H = 16
_C = 40

_TBL = 10112
_NW = 32
_EBLK = 128
_NBLK = 80
_EPAD = _NW * _NBLK * _EBLK
_NBUF = 8
_SD = 4
_NTILE = 16
_STRIPE = _TBL // _NTILE
_D2 = 48


def _sc_mesh():
    return plsc.VectorSubcoreMesh(core_axis_name="c", subcore_axis_name="s")


_SC_PARAMS = pltpu.CompilerParams(use_tc_tiling_on_sc=False)


def _deg_call(dst_idx, zeros_tbl, ones_blk):
    @functools.partial(
        pl.kernel,
        out_type=jax.ShapeDtypeStruct((2, _TBL), jnp.float32),
        mesh=_sc_mesh(),
        scratch_types=[
            pltpu.VMEM((_NBLK, _EBLK), jnp.int32),
            pltpu.VMEM((_EBLK,), jnp.float32),
            pltpu.VMEM_SHARED((_TBL,), jnp.float32),
        ]
        + [pltpu.SemaphoreType.DMA] * 8,
        compiler_params=_SC_PARAMS,
    )
    def deg_kernel(dst_hbm, zeros_hbm, ones_hbm, out_hbm, idx_v, ones_v,
                   acc_sh, *sems):
        c = lax.axis_index("c")
        s = lax.axis_index("s")
        wid = s * 2 + c
        pltpu.sync_copy(dst_hbm.at[wid], idx_v)
        pltpu.sync_copy(ones_hbm, ones_v)
        pltpu.sync_copy(
            zeros_hbm.at[pl.ds(s * _STRIPE, _STRIPE)],
            acc_sh.at[pl.ds(s * _STRIPE, _STRIPE)],
        )
        plsc.subcore_barrier()

        def body(i, carry):
            base = i * 8
            for b in range(8):
                pltpu.async_copy(ones_v, acc_sh.at[idx_v.at[base + b]],
                                 sems[b], add=True)
            for b in range(8):
                pltpu.make_async_copy(
                    ones_v, acc_sh.at[idx_v.at[base + b]], sems[b]
                ).wait()
            return carry

        lax.fori_loop(0, _NBLK // 8, body, 0)
        plsc.subcore_barrier()
        pltpu.sync_copy(
            acc_sh.at[pl.ds(s * _STRIPE, _STRIPE)],
            out_hbm.at[c, pl.ds(s * _STRIPE, _STRIPE)],
        )

    return deg_kernel(dst_idx, zeros_tbl, ones_blk)


def _agg_call(g_tbl, src_idx, dst_idx, zeros_tbl, d):
    @functools.partial(
        pl.kernel,
        out_type=jax.ShapeDtypeStruct((2, _TBL, d), jnp.float32),
        mesh=_sc_mesh(),
        scratch_types=[
            pltpu.VMEM((_NBLK, _EBLK), jnp.int32),
            pltpu.VMEM((_NBLK, _EBLK), jnp.int32),
            pltpu.VMEM((_NBUF, _EBLK, d), jnp.float32),
            pltpu.VMEM_SHARED((_TBL, d), jnp.float32),
        ]
        + [pltpu.SemaphoreType.DMA] * (2 * _NBUF),
        compiler_params=_SC_PARAMS,
    )
    def agg_kernel(g_hbm, src_hbm, dst_hbm, zeros_hbm, out_hbm,
                   src_v, dst_v, rows_v, acc_sh, *sems):
        sg = sems[:_NBUF]
        ss = sems[_NBUF:]
        c = lax.axis_index("c")
        s = lax.axis_index("s")
        wid = s * 2 + c
        pltpu.sync_copy(src_hbm.at[wid], src_v)
        pltpu.sync_copy(dst_hbm.at[wid], dst_v)
        pltpu.sync_copy(
            zeros_hbm.at[pl.ds(s * _STRIPE, _STRIPE)],
            acc_sh.at[pl.ds(s * _STRIPE, _STRIPE)],
        )
        plsc.subcore_barrier()

        for b in range(_NBUF):
            pltpu.async_copy(g_hbm.at[src_v.at[b]], rows_v.at[b], sg[b])

        def body(i, carry):
            base = i * _NBUF
            for b in range(_NBUF):
                j = base + b
                js = j - _SD
                sb = (b - _SD) % _NBUF

                @pl.when(js >= 0)
                def _():
                    pltpu.make_async_copy(
                        rows_v.at[sb], acc_sh.at[dst_v.at[js]], ss[sb]
                    ).wait()
                    nxtg = js + _NBUF

                    @pl.when(nxtg < _NBLK)
                    def _():
                        pltpu.async_copy(g_hbm.at[src_v.at[nxtg]],
                                         rows_v.at[sb], sg[sb])

                pltpu.make_async_copy(
                    g_hbm.at[src_v.at[j]], rows_v.at[b], sg[b]
                ).wait()
                pltpu.async_copy(rows_v.at[b], acc_sh.at[dst_v.at[j]],
                                 ss[b], add=True)

            return carry

        lax.fori_loop(0, _NBLK // _NBUF, body, 0)
        for b in range(_NBUF - _SD, _NBUF):
            j = _NBLK - _NBUF + b
            pltpu.make_async_copy(
                rows_v.at[b], acc_sh.at[dst_v.at[j]], ss[b]
            ).wait()
        plsc.subcore_barrier()
        pltpu.sync_copy(
            acc_sh.at[pl.ds(s * _STRIPE, _STRIPE)],
            out_hbm.at[c, pl.ds(s * _STRIPE, _STRIPE)],
        )

    return agg_kernel(g_tbl, src_idx, dst_idx, zeros_tbl)


_RBLK = 1264


def _prep1_body(d0_ref, d1_ref, x_ref, w1_ref, g1_ref, dis_ref):
    deg = d0_ref[...] + d1_ref[...] + 1.0
    dis = lax.rsqrt(deg)
    h = jnp.dot(x_ref[...], w1_ref[...], preferred_element_type=jnp.float32)
    g1_ref[...] = h * dis
    dis_ref[...] = dis


def _prep1(d0, d1, xp, w1):
    grid = (_TBL // _RBLK,)
    return pl.pallas_call(
        _prep1_body,
        grid=grid,
        in_specs=[
            pl.BlockSpec((_RBLK, 1), lambda i: (i, 0)),
            pl.BlockSpec((_RBLK, 1), lambda i: (i, 0)),
            pl.BlockSpec((_RBLK, _F), lambda i: (i, 0)),
            pl.BlockSpec((_F, _H), lambda i: (0, 0)),
        ],
        out_specs=[
            pl.BlockSpec((_RBLK, _H), lambda i: (i, 0)),
            pl.BlockSpec((_RBLK, 1), lambda i: (i, 0)),
        ],
        out_shape=[
            jax.ShapeDtypeStruct((_TBL, _H), jnp.float32),
            jax.ShapeDtypeStruct((_TBL, 1), jnp.float32),
        ],
    )(d0, d1, xp, w1)


def _prep2_body(p0_ref, p1_ref, g1_ref, dis_ref, b1_ref, u_ref):
    dis = dis_ref[...]
    a = (p0_ref[...] + p1_ref[...] + g1_ref[...]) * dis + b1_ref[...]
    u_ref[...] = jnp.maximum(a, 0.0) * dis


def _prep2(p0, p1, g1, dis, b1r):
    grid = (_TBL // _RBLK,)
    return pl.pallas_call(
        _prep2_body,
        grid=grid,
        in_specs=[
            pl.BlockSpec((_RBLK, _H), lambda i: (i, 0)),
            pl.BlockSpec((_RBLK, _H), lambda i: (i, 0)),
            pl.BlockSpec((_RBLK, _H), lambda i: (i, 0)),
            pl.BlockSpec((_RBLK, 1), lambda i: (i, 0)),
            pl.BlockSpec((1, _H), lambda i: (0, 0)),
        ],
        out_specs=pl.BlockSpec((_RBLK, _H), lambda i: (i, 0)),
        out_shape=jax.ShapeDtypeStruct((_TBL, _H), jnp.float32),
    )(p0, p1, g1, dis, b1r)


def _final_body(q0_ref, q1_ref, u_ref, dis_ref, w2_ref, b2_ref, out_ref):
    a = (q0_ref[...] + q1_ref[...] + u_ref[...]) * dis_ref[...]
    o = jnp.dot(a, w2_ref[...], preferred_element_type=jnp.float32)
    o = o + b2_ref[...]
    m = jnp.max(o, axis=1, keepdims=True)
    e = jnp.exp(o - m)
    lse = jnp.log(jnp.sum(e, axis=1, keepdims=True)) + m
    out_ref[...] = o - lse


def _final(q0, q1, u, dis, w2, b2r):
    grid = (_TBL // _RBLK,)
    return pl.pallas_call(
        _final_body,
        grid=grid,
        in_specs=[
            pl.BlockSpec((_RBLK, _H), lambda i: (i, 0)),
            pl.BlockSpec((_RBLK, _H), lambda i: (i, 0)),
            pl.BlockSpec((_RBLK, _H), lambda i: (i, 0)),
            pl.BlockSpec((_RBLK, 1), lambda i: (i, 0)),
            pl.BlockSpec((_H, _C), lambda i: (0, 0)),
            pl.BlockSpec((1, _C), lambda i: (0, 0)),
        ],
        out_specs=pl.BlockSpec((_RBLK, _C), lambda i: (i, 0)),
        out_shape=jax.ShapeDtypeStruct((_TBL, _C), jnp.float32),
    )(q0, q1, u, dis, w2, b2r)


def kernel(x, edge_index, W1, b1, W2, b2):
    pad_e = _EPAD - _E
    src = edge_index[0]
    dst = edge_index[1]
    srcp = jnp.concatenate(
        [src, jnp.zeros((pad_e,), jnp.int32)]).reshape(_NW, _NBLK, _EBLK)
    dstp = jnp.concatenate(
        [dst, jnp.full((pad_e,), _N, jnp.int32)]).reshape(_NW, _NBLK, _EBLK)
    zeros1 = jnp.zeros((_TBL,), jnp.float32)
    zeros16 = jnp.zeros((_TBL, _H), jnp.float32)
    ones_blk = jnp.ones((_EBLK,), jnp.float32)
    x_pad = jnp.pad(x, ((0, _TBL - _N), (0, 0)))

    degs = _deg_call(dstp, zeros1, ones_blk)
    d0 = degs[0].reshape(_TBL, 1)
    d1 = degs[1].reshape(_TBL, 1)

    g1, dis = _prep1(d0, d1, x_pad, W1)
    p = _agg_call(g1, srcp, dstp, zeros16, _H)
    u = _prep2(p[0], p[1], g1, dis, b1.reshape(1, _H))
    q = _agg_call(u, srcp, dstp, zeros16, _H)
    out = _final(q[0], q[1], u, dis, W2, b2.reshape(1, _C))
    return out[:_N]


# speedup vs baseline: 1.3203x; 1.3203x over previous
import functools

import jax
import jax.numpy as jnp
from jax import lax
from jax.experimental import pallas as pl
from jax.experimental.pallas import tpu as pltpu
from jax.experimental.pallas import tpu_sc as plsc

_N = 10000
_E = 320000
_F = 128
_H = 16
_C = 40

_TBL = 10112
_EBLK = 128
_NBLKS = _E // _EBLK
_NSTG = 116
_FAST_CORE = 0
_NFAST = 116
_NSLOW = 40
_SLOW_TOTAL = 644
_NBUF = 8
_NTILE = 16
_STRIPE = _TBL // _NTILE


def _sc_mesh():
    return plsc.VectorSubcoreMesh(core_axis_name="c", subcore_axis_name="s")


_SC_PARAMS = pltpu.CompilerParams(use_tc_tiling_on_sc=False)


def _worker_span(c, s):
    is_fast = c == _FAST_CORE
    slow_start = s * _NSLOW + jnp.minimum(s, 4)
    fast_start = _SLOW_TOTAL + s * _NFAST
    start = jnp.where(is_fast, fast_start, slow_start)
    cnt = jnp.where(is_fast, _NFAST,
                    _NSLOW + jnp.where(s < 4, 1, 0))
    return start, cnt


def _deg_call(dst_blocks, zeros_col, ones_col):
    @functools.partial(
        pl.kernel,
        out_type=(
            jax.ShapeDtypeStruct((_TBL, 1), jnp.float32),
            jax.ShapeDtypeStruct((_TBL, 1), jnp.float32),
        ),
        mesh=_sc_mesh(),
        scratch_types=[
            pltpu.VMEM((_NSTG, _EBLK), jnp.int32),
            pltpu.VMEM((_EBLK, 1), jnp.float32),
            pltpu.VMEM_SHARED((_TBL, 1), jnp.float32),
            pltpu.SemaphoreType.DMA,
        ],
        compiler_params=_SC_PARAMS,
    )
    def deg_kernel(dst_hbm, zeros_hbm, ones_hbm, out0_hbm, out1_hbm,
                   idx_v, ones_v, acc_sh, sem):
        c = lax.axis_index("c")
        s = lax.axis_index("s")
        start, cnt = _worker_span(c, s)
        pltpu.sync_copy(dst_hbm.at[pl.ds(start, _NSTG)], idx_v)
        pltpu.sync_copy(ones_hbm, ones_v)
        pltpu.sync_copy(
            zeros_hbm.at[pl.ds(s * _STRIPE, _STRIPE)],
            acc_sh.at[pl.ds(s * _STRIPE, _STRIPE)],
        )
        plsc.subcore_barrier()

        def wave(iw, carry):
            base = iw * _NBUF
            for b in range(_NBUF):
                @pl.when(base + b < cnt)
                def _():
                    pltpu.async_copy(ones_v, acc_sh.at[idx_v.at[base + b]],
                                     sem, add=True)
            for b in range(_NBUF):
                @pl.when(base + b < cnt)
                def _():
                    pltpu.make_async_copy(
                        ones_v, acc_sh.at[idx_v.at[base + b]], sem
                    ).wait()
            return carry

        lax.fori_loop(0, (_NSTG + _NBUF - 1) // _NBUF, wave, 0)
        plsc.subcore_barrier()
        row = pl.ds(s * _STRIPE, _STRIPE)

        @pl.when(c == 0)
        def _():
            pltpu.sync_copy(acc_sh.at[row], out0_hbm.at[row])

        @pl.when(c == 1)
        def _():
            pltpu.sync_copy(acc_sh.at[row], out1_hbm.at[row])

    return deg_kernel(dst_blocks, zeros_col, ones_col)


def _agg_call(g_tbl, src_blocks, dst_blocks, zeros_tbl):
    @functools.partial(
        pl.kernel,
        out_type=jax.ShapeDtypeStruct((2, _TBL, _H), jnp.float32),
        mesh=_sc_mesh(),
        scratch_types=[
            pltpu.VMEM((_NSTG, _EBLK), jnp.int32),
            pltpu.VMEM((_NSTG, _EBLK), jnp.int32),
            pltpu.VMEM((_NBUF, _EBLK, _H), jnp.float32),
            pltpu.VMEM_SHARED((_TBL, _H), jnp.float32),
            pltpu.SemaphoreType.DMA,
            pltpu.SemaphoreType.DMA,
        ],
        compiler_params=_SC_PARAMS,
    )
    def agg_kernel(g_hbm, src_hbm, dst_hbm, zeros_hbm, out_hbm,
                   src_v, dst_v, rows_v, acc_sh, sem_g, sem_s):
        c = lax.axis_index("c")
        s = lax.axis_index("s")
        start, cnt = _worker_span(c, s)
        pltpu.sync_copy(src_hbm.at[pl.ds(start, _NSTG)], src_v)
        pltpu.sync_copy(dst_hbm.at[pl.ds(start, _NSTG)], dst_v)
        pltpu.sync_copy(
            zeros_hbm.at[pl.ds(s * _STRIPE, _STRIPE)],
            acc_sh.at[pl.ds(s * _STRIPE, _STRIPE)],
        )
        plsc.subcore_barrier()

        def wave(iw, carry):
            base = iw * _NBUF
            for b in range(_NBUF):
                @pl.when(base + b < cnt)
                def _():
                    pltpu.async_copy(g_hbm.at[src_v.at[base + b]],
                                     rows_v.at[b], sem_g)
            for b in range(_NBUF):
                @pl.when(base + b < cnt)
                def _():
                    pltpu.make_async_copy(
                        g_hbm.at[src_v.at[base + b]], rows_v.at[b], sem_g
                    ).wait()
            for b in range(_NBUF):
                @pl.when(base + b < cnt)
                def _():
                    pltpu.async_copy(rows_v.at[b], acc_sh.at[dst_v.at[base + b]],
                                     sem_s, add=True)
            for b in range(_NBUF):
                @pl.when(base + b < cnt)
                def _():
                    pltpu.make_async_copy(
                        rows_v.at[b], acc_sh.at[dst_v.at[base + b]], sem_s
                    ).wait()
            return carry

        lax.fori_loop(0, (_NSTG + _NBUF - 1) // _NBUF, wave, 0)
        plsc.subcore_barrier()
        row = pl.ds(s * _STRIPE, _STRIPE)
        pltpu.sync_copy(acc_sh.at[row], out_hbm.at[c, row])

    return agg_kernel(g_tbl, src_blocks, dst_blocks, zeros_tbl)


_RBLK = 1264


def _prep1_body(d0_ref, d1_ref, x_ref, w1_ref, g1_ref, dis_ref):
    deg = d0_ref[...] + d1_ref[...] + 1.0
    dis = lax.rsqrt(deg)
    h = jnp.dot(x_ref[...], w1_ref[...], preferred_element_type=jnp.float32)
    g1_ref[...] = h * dis
    dis_ref[...] = dis


def _prep1(d0, d1, x, w1):
    grid = (_TBL // _RBLK,)
    return pl.pallas_call(
        _prep1_body,
        grid=grid,
        in_specs=[
            pl.BlockSpec((_RBLK, 1), lambda i: (i, 0)),
            pl.BlockSpec((_RBLK, 1), lambda i: (i, 0)),
            pl.BlockSpec((_RBLK, _F), lambda i: (i, 0)),
            pl.BlockSpec((_F, _H), lambda i: (0, 0)),
        ],
        out_specs=[
            pl.BlockSpec((_RBLK, _H), lambda i: (i, 0)),
            pl.BlockSpec((_RBLK, 1), lambda i: (i, 0)),
        ],
        out_shape=[
            jax.ShapeDtypeStruct((_TBL, _H), jnp.float32),
            jax.ShapeDtypeStruct((_TBL, 1), jnp.float32),
        ],
    )(d0, d1, x, w1)


def _prep2_body(p0_ref, p1_ref, g1_ref, dis_ref, b1_ref, u_ref):
    dis = dis_ref[...]
    a = (p0_ref[0] + p1_ref[0] + g1_ref[...]) * dis + b1_ref[...]
    u_ref[...] = jnp.maximum(a, 0.0) * dis


def _prep2(p, g1, dis, b1r):
    grid = (_TBL // _RBLK,)
    return pl.pallas_call(
        _prep2_body,
        grid=grid,
        in_specs=[
            pl.BlockSpec((1, _RBLK, _H), lambda i: (0, i, 0)),
            pl.BlockSpec((1, _RBLK, _H), lambda i: (1, i, 0)),
            pl.BlockSpec((_RBLK, _H), lambda i: (i, 0)),
            pl.BlockSpec((_RBLK, 1), lambda i: (i, 0)),
            pl.BlockSpec((1, _H), lambda i: (0, 0)),
        ],
        out_specs=pl.BlockSpec((_RBLK, _H), lambda i: (i, 0)),
        out_shape=jax.ShapeDtypeStruct((_TBL, _H), jnp.float32),
    )(p, p, g1, dis, b1r)


def _final_body(q0_ref, q1_ref, u_ref, dis_ref, w2_ref, b2_ref, out_ref):
    a = (q0_ref[0] + q1_ref[0] + u_ref[...]) * dis_ref[...]
    o = jnp.dot(a, w2_ref[...], preferred_element_type=jnp.float32)
    o = o + b2_ref[...]
    m = jnp.max(o, axis=1, keepdims=True)
    e = jnp.exp(o - m)
    lse = jnp.log(jnp.sum(e, axis=1, keepdims=True)) + m
    out_ref[...] = o - lse


def _final(q, u, dis, w2, b2r):
    grid = (_TBL // _RBLK,)
    return pl.pallas_call(
        _final_body,
        grid=grid,
        in_specs=[
            pl.BlockSpec((1, _RBLK, _H), lambda i: (0, i, 0)),
            pl.BlockSpec((1, _RBLK, _H), lambda i: (1, i, 0)),
            pl.BlockSpec((_RBLK, _H), lambda i: (i, 0)),
            pl.BlockSpec((_RBLK, 1), lambda i: (i, 0)),
            pl.BlockSpec((_H, _C), lambda i: (0, 0)),
            pl.BlockSpec((1, _C), lambda i: (0, 0)),
        ],
        out_specs=pl.BlockSpec((_RBLK, _C), lambda i: (i, 0)),
        out_shape=jax.ShapeDtypeStruct((_TBL, _C), jnp.float32),
    )(q, q, u, dis, w2, b2r)


def kernel(x, edge_index, W1, b1, W2, b2):
    eb = edge_index.reshape(2, _NBLKS, _EBLK)
    srcb = eb[0]
    dstb = eb[1]
    zeros_col = jnp.zeros((_TBL, 1), jnp.float32)
    zeros16 = jnp.zeros((_TBL, _H), jnp.float32)
    ones_col = jnp.ones((_EBLK, 1), jnp.float32)

    d0, d1 = _deg_call(dstb, zeros_col, ones_col)
    g1, dis = _prep1(d0, d1, x, W1)
    p = _agg_call(g1, srcb, dstb, zeros16)
    u = _prep2(p, g1, dis, b1.reshape(1, _H))
    q = _agg_call(u, srcb, dstb, zeros16)
    out = _final(q, u, dis, W2, b2.reshape(1, _C))
    return out[:_N]


# speedup vs baseline: 1.4839x; 1.1239x over previous
import functools

import jax
import jax.numpy as jnp
from jax import lax
from jax.experimental import pallas as pl
from jax.experimental.pallas import tpu as pltpu
from jax.experimental.pallas import tpu_sc as plsc

_N = 10000
_E = 320000
_F = 128
_H = 16
_C = 40

_TBL = 10112
_EBLK = 128
_NBLKS = _E // _EBLK
_NSTG = 116
_FAST_CORE = 0
_NFAST = 116
_NSLOW = 40
_SLOW_TOTAL = 644
_NBUF = 8
_SD = 4
_NTILE = 16
_STRIPE = _TBL // _NTILE


def _sc_mesh():
    return plsc.VectorSubcoreMesh(core_axis_name="c", subcore_axis_name="s")


_SC_PARAMS = pltpu.CompilerParams(use_tc_tiling_on_sc=False)


def _worker_span(c, s):
    is_fast = c == _FAST_CORE
    slow_start = s * _NSLOW + jnp.minimum(s, 4)
    fast_start = _SLOW_TOTAL + s * _NFAST
    start = jnp.where(is_fast, fast_start, slow_start)
    cnt = jnp.where(is_fast, _NFAST,
                    _NSLOW + jnp.where(s < 4, 1, 0))
    return start, cnt


def _deg_call(dstb, zeros_col, ones_col):
    @functools.partial(
        pl.kernel,
        out_type=(
            jax.ShapeDtypeStruct((_TBL, 1), jnp.float32),
            jax.ShapeDtypeStruct((_TBL, 1), jnp.float32),
        ),
        mesh=_sc_mesh(),
        scratch_types=[
            pltpu.VMEM((_NSTG, _EBLK), jnp.int32),
            pltpu.VMEM((_EBLK, 1), jnp.float32),
            pltpu.VMEM_SHARED((_TBL, 1), jnp.float32),
        ]
        + [pltpu.SemaphoreType.DMA] * _NBUF,
        compiler_params=_SC_PARAMS,
    )
    def deg_kernel(dst_hbm, zeros_hbm, ones_hbm, out0_hbm, out1_hbm,
                   idx_v, ones_v, acc_sh, *ss):
        c = lax.axis_index("c")
        s = lax.axis_index("s")
        start, cnt = _worker_span(c, s)
        pltpu.sync_copy(dst_hbm.at[pl.ds(start, _NSTG)], idx_v)
        pltpu.sync_copy(ones_hbm, ones_v)
        pltpu.sync_copy(
            zeros_hbm.at[pl.ds(s * _STRIPE, _STRIPE)],
            acc_sh.at[pl.ds(s * _STRIPE, _STRIPE)],
        )
        plsc.subcore_barrier()

        def body(i, carry):
            base = i * _NBUF
            for b in range(_NBUF):
                j = base + b
                jp = j - _NBUF

                @pl.when((jp >= 0) & (jp < cnt))
                def _():
                    pltpu.make_async_copy(
                        ones_v, acc_sh.at[idx_v.at[jp]], ss[b]
                    ).wait()

                @pl.when(j < cnt)
                def _():
                    pltpu.async_copy(ones_v, acc_sh.at[idx_v.at[j]],
                                     ss[b], add=True)

            return carry

        lax.fori_loop(0, (_NSTG + 2 * _NBUF - 1) // _NBUF, body, 0)
        plsc.subcore_barrier()
        row = pl.ds(s * _STRIPE, _STRIPE)

        @pl.when(c == 0)
        def _():
            pltpu.sync_copy(acc_sh.at[row], out0_hbm.at[row])

        @pl.when(c == 1)
        def _():
            pltpu.sync_copy(acc_sh.at[row], out1_hbm.at[row])

    return deg_kernel(dstb, zeros_col, ones_col)


def _agg_call(g_tbl, srcb, dstb, zeros_tbl):
    @functools.partial(
        pl.kernel,
        out_type=jax.ShapeDtypeStruct((2, _TBL, _H), jnp.float32),
        mesh=_sc_mesh(),
        scratch_types=[
            pltpu.VMEM((_NSTG, _EBLK), jnp.int32),
            pltpu.VMEM((_NSTG, _EBLK), jnp.int32),
            pltpu.VMEM((_NBUF, _EBLK, _H), jnp.float32),
            pltpu.VMEM_SHARED((_TBL, _H), jnp.float32),
        ]
        + [pltpu.SemaphoreType.DMA] * (2 * _NBUF),
        compiler_params=_SC_PARAMS,
    )
    def agg_kernel(g_hbm, src_hbm, dst_hbm, zeros_hbm, out_hbm,
                   src_v, dst_v, rows_v, acc_sh, *sems):
        sg = sems[:_NBUF]
        ss = sems[_NBUF:]
        c = lax.axis_index("c")
        s = lax.axis_index("s")
        start, cnt = _worker_span(c, s)
        pltpu.sync_copy(src_hbm.at[pl.ds(start, _NSTG)], src_v)
        pltpu.sync_copy(dst_hbm.at[pl.ds(start, _NSTG)], dst_v)
        pltpu.sync_copy(
            zeros_hbm.at[pl.ds(s * _STRIPE, _STRIPE)],
            acc_sh.at[pl.ds(s * _STRIPE, _STRIPE)],
        )
        plsc.subcore_barrier()

        for b in range(_NBUF):
            pltpu.async_copy(g_hbm.at[src_v.at[b]], rows_v.at[b], sg[b])

        def body(i, carry):
            base = i * _NBUF
            for b in range(_NBUF):
                j = base + b
                js = j - _SD
                sb = (b - _SD) % _NBUF

                @pl.when((js >= 0) & (js < cnt))
                def _():
                    pltpu.make_async_copy(
                        rows_v.at[sb], acc_sh.at[dst_v.at[js]], ss[sb]
                    ).wait()
                    nxt = js + _NBUF

                    @pl.when(nxt < cnt)
                    def _():
                        pltpu.async_copy(g_hbm.at[src_v.at[nxt]],
                                         rows_v.at[sb], sg[sb])

                @pl.when(j < cnt)
                def _():
                    pltpu.make_async_copy(
                        g_hbm.at[src_v.at[j]], rows_v.at[b], sg[b]
                    ).wait()
                    pltpu.async_copy(rows_v.at[b], acc_sh.at[dst_v.at[j]],
                                     ss[b], add=True)

            return carry

        lax.fori_loop(0, (_NSTG + _SD + _NBUF - 1) // _NBUF, body, 0)
        plsc.subcore_barrier()
        row = pl.ds(s * _STRIPE, _STRIPE)
        pltpu.sync_copy(acc_sh.at[row], out_hbm.at[c, row])

    return agg_kernel(g_tbl, srcb, dstb, zeros_tbl)


_RBLK = 1264


def _h1_body(x_ref, w1_ref, h1_ref):
    h1_ref[...] = jnp.dot(x_ref[...], w1_ref[...],
                          preferred_element_type=jnp.float32)


def _h1_call(x, w1):
    grid = (_TBL // _RBLK,)
    return pl.pallas_call(
        _h1_body,
        grid=grid,
        in_specs=[
            pl.BlockSpec((_RBLK, _F), lambda i: (i, 0)),
            pl.BlockSpec((_F, _H), lambda i: (0, 0)),
        ],
        out_specs=pl.BlockSpec((_RBLK, _H), lambda i: (i, 0)),
        out_shape=jax.ShapeDtypeStruct((_TBL, _H), jnp.float32),
    )(x, w1)


def _prep1_body(d0_ref, d1_ref, h1_ref, g1_ref, dis_ref):
    deg = d0_ref[...] + d1_ref[...] + 1.0
    dis = lax.rsqrt(deg)
    g1_ref[...] = h1_ref[...] * dis
    dis_ref[...] = dis


def _prep1(d0, d1, h1):
    grid = (_TBL // _RBLK,)
    return pl.pallas_call(
        _prep1_body,
        grid=grid,
        in_specs=[
            pl.BlockSpec((_RBLK, 1), lambda i: (i, 0)),
            pl.BlockSpec((_RBLK, 1), lambda i: (i, 0)),
            pl.BlockSpec((_RBLK, _H), lambda i: (i, 0)),
        ],
        out_specs=[
            pl.BlockSpec((_RBLK, _H), lambda i: (i, 0)),
            pl.BlockSpec((_RBLK, 1), lambda i: (i, 0)),
        ],
        out_shape=[
            jax.ShapeDtypeStruct((_TBL, _H), jnp.float32),
            jax.ShapeDtypeStruct((_TBL, 1), jnp.float32),
        ],
    )(d0, d1, h1)


def _prep2_body(p0_ref, p1_ref, g1_ref, dis_ref, b1_ref, u_ref):
    dis = dis_ref[...]
    a = (p0_ref[0] + p1_ref[0] + g1_ref[...]) * dis + b1_ref[...]
    u_ref[...] = jnp.maximum(a, 0.0) * dis


def _prep2(p, g1, dis, b1r):
    grid = (_TBL // _RBLK,)
    return pl.pallas_call(
        _prep2_body,
        grid=grid,
        in_specs=[
            pl.BlockSpec((1, _RBLK, _H), lambda i: (0, i, 0)),
            pl.BlockSpec((1, _RBLK, _H), lambda i: (1, i, 0)),
            pl.BlockSpec((_RBLK, _H), lambda i: (i, 0)),
            pl.BlockSpec((_RBLK, 1), lambda i: (i, 0)),
            pl.BlockSpec((1, _H), lambda i: (0, 0)),
        ],
        out_specs=pl.BlockSpec((_RBLK, _H), lambda i: (i, 0)),
        out_shape=jax.ShapeDtypeStruct((_TBL, _H), jnp.float32),
    )(p, p, g1, dis, b1r)


def _final_body(q0_ref, q1_ref, u_ref, dis_ref, w2_ref, b2_ref, out_ref):
    a = (q0_ref[0] + q1_ref[0] + u_ref[...]) * dis_ref[...]
    o = jnp.dot(a, w2_ref[...], preferred_element_type=jnp.float32)
    o = o + b2_ref[...]
    m = jnp.max(o, axis=1, keepdims=True)
    e = jnp.exp(o - m)
    lse = jnp.log(jnp.sum(e, axis=1, keepdims=True)) + m
    out_ref[...] = o - lse


def _final(q, u, dis, w2, b2r):
    grid = (_TBL // _RBLK,)
    return pl.pallas_call(
        _final_body,
        grid=grid,
        in_specs=[
            pl.BlockSpec((1, _RBLK, _H), lambda i: (0, i, 0)),
            pl.BlockSpec((1, _RBLK, _H), lambda i: (1, i, 0)),
            pl.BlockSpec((_RBLK, _H), lambda i: (i, 0)),
            pl.BlockSpec((_RBLK, 1), lambda i: (i, 0)),
            pl.BlockSpec((_H, _C), lambda i: (0, 0)),
            pl.BlockSpec((1, _C), lambda i: (0, 0)),
        ],
        out_specs=pl.BlockSpec((_RBLK, _C), lambda i: (i, 0)),
        out_shape=jax.ShapeDtypeStruct((_TBL, _C), jnp.float32),
    )(q, q, u, dis, w2, b2r)


def kernel(x, edge_index, W1, b1, W2, b2):
    eb = edge_index.reshape(2, _NBLKS, _EBLK)
    srcb = eb[0]
    dstb = eb[1]
    zeros_col = jnp.zeros((_TBL, 1), jnp.float32)
    zeros16 = jnp.zeros((_TBL, _H), jnp.float32)
    ones_col = jnp.ones((_EBLK, 1), jnp.float32)

    d0, d1 = _deg_call(dstb, zeros_col, ones_col)
    h1 = _h1_call(x, W1)
    g1, dis = _prep1(d0, d1, h1)
    p = _agg_call(g1, srcb, dstb, zeros16)
    u = _prep2(p, g1, dis, b1.reshape(1, _H))
    q = _agg_call(u, srcb, dstb, zeros16)
    out = _final(q, u, dis, W2, b2.reshape(1, _C))
    return out[:_N]
